# baseline (speedup 1.0000x reference)
import functools

import jax
import jax.numpy as jnp
from jax import lax
from jax.experimental import pallas as pl
from jax.experimental.pallas import tpu as pltpu
from jax.experimental.pallas import tpu_sc as plsc

_NUM_CORES = 2
_NUM_SUBCORES = 16
_NUM_WORKERS = _NUM_CORES * _NUM_SUBCORES
_CHUNK = 128


def _mesh():
    return plsc.VectorSubcoreMesh(
        core_axis_name="c", subcore_axis_name="s",
        num_cores=_NUM_CORES, num_subcores=_NUM_SUBCORES,
    )


def _worker_id():
    return lax.axis_index("s") * _NUM_CORES + lax.axis_index("c")


def _fast(batch, seq, win, head, dtype, kv, kv_cache):
    b_per_w = batch // _NUM_WORKERS

    @functools.partial(
        pl.kernel,
        out_type=jax.ShapeDtypeStruct((batch, win, head), dtype),
        mesh=_mesh(),
    )
    def body(kv_hbm, cache_hbm, out_hbm):
        w = _worker_id()
        for i in range(b_per_w):
            b = w * b_per_w + i
            pltpu.sync_copy(kv_hbm.at[b], out_hbm.at[b, pl.ds(0, seq)])
            pltpu.sync_copy(
                cache_hbm.at[b, pl.ds(seq, win - seq)],
                out_hbm.at[b, pl.ds(seq, win - seq)],
            )

    return body(kv, kv_cache)


def _general(batch, seq, win, head, dtype, kv, kv_cache, pos):
    b_per_w = batch // _NUM_WORKERS
    n_chunks = seq // _CHUNK
    h32 = head // 2
    idx = (jnp.arange(batch, dtype=jnp.int32)[:, None] * win
           + pos + jnp.arange(seq, dtype=jnp.int32)[None, :])
    idx = idx.reshape(batch, n_chunks, _CHUNK)
    kv_i32 = lax.bitcast_convert_type(
        kv.reshape(batch * seq, h32, 2), jnp.int32)
    cache_i32 = lax.bitcast_convert_type(
        kv_cache.reshape(batch, win, h32, 2), jnp.int32)

    @functools.partial(
        pl.kernel,
        out_type=jax.ShapeDtypeStruct((batch * win, h32), jnp.int32),
        mesh=_mesh(),
        scratch_types=[
            pltpu.VMEM((_CHUNK,), jnp.int32),
            pltpu.VMEM((_CHUNK, h32), jnp.int32),
            pltpu.SemaphoreType.DMA,
        ],
        compiler_params=pltpu.CompilerParams(use_tc_tiling_on_sc=False),
    )
    def body(kv_hbm, cache_hbm, idx_hbm, out_hbm, idx_v, rows_v, sem):
        w = _worker_id()
        for i in range(b_per_w):
            b = w * b_per_w + i
            pltpu.sync_copy(
                cache_hbm.at[b],
                out_hbm.at[pl.ds(b * win, win)],
            )
            for c in range(n_chunks):
                pltpu.sync_copy(idx_hbm.at[b, c], idx_v)
                pltpu.sync_copy(
                    kv_hbm.at[pl.ds(b * seq + c * _CHUNK, _CHUNK)], rows_v)
                pltpu.async_copy(rows_v, out_hbm.at[idx_v], sem).wait()

    out = body(kv_i32, cache_i32, idx)
    out = lax.bitcast_convert_type(out, dtype)
    return out.reshape(batch, win, head)


def kernel(kv, kv_cache, start_pos):
    batch, seq, head = kv.shape
    win = kv_cache.shape[1]
    dtype = kv_cache.dtype
    pos = jnp.asarray(start_pos, jnp.int32) % win
    pos = jnp.minimum(pos, win - seq)
    return lax.cond(
        pos == 0,
        lambda a, b: _fast(batch, seq, win, head, dtype, a, b),
        lambda a, b: _general(batch, seq, win, head, dtype, a, b, pos),
        kv, kv_cache,
    )


# speedup vs baseline: 35.7574x; 35.7574x over previous
import functools

import jax
import jax.numpy as jnp
from jax import lax
from jax.experimental import pallas as pl
from jax.experimental.pallas import tpu as pltpu
from jax.experimental.pallas import tpu_sc as plsc

_NUM_CORES = 2
_NUM_SUBCORES = 16
_NUM_WORKERS = _NUM_CORES * _NUM_SUBCORES
_CHUNK = 128


def _mesh():
    return plsc.VectorSubcoreMesh(
        core_axis_name="c", subcore_axis_name="s",
        num_cores=_NUM_CORES, num_subcores=_NUM_SUBCORES,
    )


def _worker_id():
    return lax.axis_index("s") * _NUM_CORES + lax.axis_index("c")


_CH = 512


def _fast(batch, seq, win, head, dtype, kv, kv_cache):
    b_per_w = batch // _NUM_WORKERS

    @functools.partial(
        pl.kernel,
        out_type=jax.ShapeDtypeStruct((batch, win, head), dtype),
        mesh=_mesh(),
        scratch_types=[
            pltpu.VMEM((2, _CH, head), dtype),
            pltpu.SemaphoreType.DMA,
            pltpu.SemaphoreType.DMA,
            pltpu.SemaphoreType.DMA,
            pltpu.SemaphoreType.DMA,
        ],
    )
    def body(kv_hbm, cache_hbm, out_hbm, buf, in0, in1, out0, out1):
        w = _worker_id()
        in_sems = (in0, in1)
        out_sems = (out0, out1)
        ins, outs = [], []
        for i in range(b_per_w):
            b = w * b_per_w + i
            for c in range(seq // _CH):
                n = len(ins)
                ins.append(pltpu.make_async_copy(
                    kv_hbm.at[b, pl.ds(c * _CH, _CH)],
                    buf.at[n & 1], in_sems[n & 1]))
                outs.append(pltpu.make_async_copy(
                    buf.at[n & 1],
                    out_hbm.at[b, pl.ds(c * _CH, _CH)], out_sems[n & 1]))
            for c in range((win - seq) // _CH):
                n = len(ins)
                ins.append(pltpu.make_async_copy(
                    cache_hbm.at[b, pl.ds(seq + c * _CH, _CH)],
                    buf.at[n & 1], in_sems[n & 1]))
                outs.append(pltpu.make_async_copy(
                    buf.at[n & 1],
                    out_hbm.at[b, pl.ds(seq + c * _CH, _CH)],
                    out_sems[n & 1]))
        nb = len(ins)
        ins[0].start()
        for c in range(nb):
            if c + 1 < nb:
                if c >= 1:
                    outs[c - 1].wait()
                ins[c + 1].start()
            ins[c].wait()
            outs[c].start()
        if nb >= 2:
            outs[nb - 2].wait()
        outs[nb - 1].wait()

    return body(kv, kv_cache)


def _general(batch, seq, win, head, dtype, kv, kv_cache, pos):
    b_per_w = batch // _NUM_WORKERS
    n_chunks = seq // _CHUNK
    h32 = head // 2
    idx = (jnp.arange(batch, dtype=jnp.int32)[:, None] * win
           + pos + jnp.arange(seq, dtype=jnp.int32)[None, :])
    idx = idx.reshape(batch, n_chunks, _CHUNK)
    kv_i32 = lax.bitcast_convert_type(
        kv.reshape(batch * seq, h32, 2), jnp.int32)
    cache_i32 = lax.bitcast_convert_type(
        kv_cache.reshape(batch, win, h32, 2), jnp.int32)

    @functools.partial(
        pl.kernel,
        out_type=jax.ShapeDtypeStruct((batch * win, h32), jnp.int32),
        mesh=_mesh(),
        scratch_types=[
            pltpu.VMEM((_CHUNK,), jnp.int32),
            pltpu.VMEM((_CHUNK, h32), jnp.int32),
            pltpu.SemaphoreType.DMA,
        ],
        compiler_params=pltpu.CompilerParams(use_tc_tiling_on_sc=False),
    )
    def body(kv_hbm, cache_hbm, idx_hbm, out_hbm, idx_v, rows_v, sem):
        w = _worker_id()
        for i in range(b_per_w):
            b = w * b_per_w + i
            pltpu.sync_copy(
                cache_hbm.at[b],
                out_hbm.at[pl.ds(b * win, win)],
            )
            for c in range(n_chunks):
                pltpu.sync_copy(idx_hbm.at[b, c], idx_v)
                pltpu.sync_copy(
                    kv_hbm.at[pl.ds(b * seq + c * _CHUNK, _CHUNK)], rows_v)
                pltpu.async_copy(rows_v, out_hbm.at[idx_v], sem).wait()

    out = body(kv_i32, cache_i32, idx)
    out = lax.bitcast_convert_type(out, dtype)
    return out.reshape(batch, win, head)


def kernel(kv, kv_cache, start_pos):
    batch, seq, head = kv.shape
    win = kv_cache.shape[1]
    dtype = kv_cache.dtype
    pos = jnp.asarray(start_pos, jnp.int32) % win
    pos = jnp.minimum(pos, win - seq)
    return lax.cond(
        pos == 0,
        lambda a, b: _fast(batch, seq, win, head, dtype, a, b),
        lambda a, b: _general(batch, seq, win, head, dtype, a, b, pos),
        kv, kv_cache,
    )
